# Initial kernel scaffold; baseline (speedup 1.0000x reference)
import jax
import jax.numpy as jnp
from jax.experimental import pallas as pl


def kernel(x, edge_index, edge_attr, W_ne, b_ne, W_ee, b_ee, W_e, b_e, W_n, b_n, W_d1, b_d1, W_d2, b_d2, W_r, b_r):
    raise NotImplementedError("write your pallas kernel here")


# speedup vs baseline: 2.9581x; 2.9581x over previous
import functools

import jax
import jax.numpy as jnp
from jax import lax
from jax.experimental import pallas as pl
from jax.experimental.pallas import tpu as pltpu
from jax.experimental.pallas import tpu_sc as plsc

N = 10000
E = 320000
DF = 128
DE = 4
L = 128

NC = 2
NS = 16
NW = NC * NS
EPW = E // NW
C = 80
NCHUNK = EPW // C
RPS = 624

_f32 = jnp.float32


def _dot(a, b):
    return jnp.dot(a, b, preferred_element_type=_f32)


def _node_encode_body(x_ref, wne_ref, bne_ref, wa_ref, wb_ref,
                      h_ref, p_ref, q_ref):
    h = jnp.maximum(_dot(x_ref[...], wne_ref[...]) + bne_ref[...], 0.0)
    h_ref[...] = h
    p_ref[...] = _dot(h, wa_ref[...])
    q_ref[...] = _dot(h, wb_ref[...])


def _node_encode(x, W_ne, b_ne, Wa, Wb):
    return pl.pallas_call(
        _node_encode_body,
        out_shape=[jax.ShapeDtypeStruct((N, L), _f32)] * 3,
    )(x, W_ne, b_ne, Wa, Wb)


BE = 6400


def _edge_u0_body(ea_ref, wee_ref, bee_ref, wc_ref, be_ref, u_ref):
    e0 = jnp.maximum(_dot(ea_ref[...], wee_ref[...]) + bee_ref[...], 0.0)
    u_ref[...] = _dot(e0, wc_ref[...]) + be_ref[...]


def _edge_u0(edge_attr, W_ee, b_ee, Wc, be):
    return pl.pallas_call(
        _edge_u0_body,
        grid=(E // BE,),
        in_specs=[
            pl.BlockSpec((BE, DE), lambda i: (i, 0)),
            pl.BlockSpec((DE, L), lambda i: (0, 0)),
            pl.BlockSpec((1, L), lambda i: (0, 0)),
            pl.BlockSpec((L, L), lambda i: (0, 0)),
            pl.BlockSpec((1, L), lambda i: (0, 0)),
        ],
        out_specs=pl.BlockSpec((BE, L), lambda i: (i, 0)),
        out_shape=jax.ShapeDtypeStruct((E, L), _f32),
    )(edge_attr, W_ee, b_ee, Wc, be)


def _edge_u_body(e_ref, wc_ref, be_ref, u_ref):
    u_ref[...] = _dot(e_ref[...], wc_ref[...]) + be_ref[...]


def _edge_u(e, Wc, be):
    return pl.pallas_call(
        _edge_u_body,
        grid=(E // BE,),
        in_specs=[
            pl.BlockSpec((BE, L), lambda i: (i, 0)),
            pl.BlockSpec((L, L), lambda i: (0, 0)),
            pl.BlockSpec((1, L), lambda i: (0, 0)),
        ],
        out_specs=pl.BlockSpec((BE, L), lambda i: (i, 0)),
        out_shape=jax.ShapeDtypeStruct((E, L), _f32),
    )(e, Wc, be)


def _node_update_body(h_ref, a_ref, wnh_ref, wna_ref, bn_ref,
                      wa_ref, wb_ref, h1_ref, p_ref, q_ref):
    agg = a_ref[0] + a_ref[1]
    h1 = jnp.maximum(
        _dot(h_ref[...], wnh_ref[...]) + _dot(agg, wna_ref[...]) + bn_ref[...],
        0.0)
    h1_ref[...] = h1
    p_ref[...] = _dot(h1, wa_ref[...])
    q_ref[...] = _dot(h1, wb_ref[...])


def _node_update(h, aggs, Wnh, Wna, bn, Wa, Wb):
    return pl.pallas_call(
        _node_update_body,
        out_shape=[jax.ShapeDtypeStruct((N, L), _f32)] * 3,
    )(h, aggs, Wnh, Wna, bn, Wa, Wb)


def _final_body(h_ref, a_ref, wnh_ref, wna_ref, bn_ref, wd1_ref, bd1_ref,
                wd2_ref, bd2_ref, wr_ref, br_ref, out_ref):
    agg = a_ref[0] + a_ref[1]
    h2 = jnp.maximum(
        _dot(h_ref[...], wnh_ref[...]) + _dot(agg, wna_ref[...]) + bn_ref[...],
        0.0)
    d = jnp.maximum(_dot(h2, wd1_ref[...]) + bd1_ref[...], 0.0)
    d = jnp.maximum(_dot(d, wd2_ref[...]) + bd2_ref[...], 0.0)
    out_ref[...] = _dot(d, wr_ref[...]) + br_ref[...]


def _final(h, aggs, Wnh, Wna, bn, W_d1, b_d1, W_d2, b_d2, W_r, b_r):
    return pl.pallas_call(
        _final_body,
        out_shape=jax.ShapeDtypeStruct((N, 1), _f32),
    )(h, aggs, Wnh, Wna, bn, W_d1, b_d1, W_d2, b_d2, W_r, b_r)


def _make_sc_step(write_e: bool):
    mesh = plsc.VectorSubcoreMesh(core_axis_name="c", subcore_axis_name="s")
    out_type = [jax.ShapeDtypeStruct((NC, N, L), _f32)]
    if write_e:
        out_type = [jax.ShapeDtypeStruct((E, L), _f32)] + out_type

    @functools.partial(
        pl.kernel,
        mesh=mesh,
        out_type=out_type,
        scratch_types=[
            pltpu.VMEM((C,), jnp.int32),
            pltpu.VMEM((C,), jnp.int32),
            pltpu.VMEM((C, L), _f32),
            pltpu.VMEM((C, L), _f32),
            pltpu.VMEM((C, L), _f32),
            pltpu.VMEM_SHARED((N, L), _f32),
        ],
    )
    def sc_step(p_hbm, q_hbm, u_hbm, src_hbm, dst_hbm, *refs):
        if write_e:
            e_out, agg_out, idx_s, idx_d, buf_p, buf_q, buf_u, agg_sh = refs
        else:
            agg_out, idx_s, idx_d, buf_p, buf_q, buf_u, agg_sh = refs
        cid = lax.axis_index("c")
        sid = lax.axis_index("s")
        wid = sid * NC + cid
        base = wid * EPW

        def zfill(i, carry):
            for j in range(L // 16):
                buf_p[i, pl.ds(j * 16, 16)] = jnp.zeros((16,), _f32)
            return carry
        lax.fori_loop(0, C, zfill, 0)
        zbase = sid * RPS
        for z in range(RPS // C):
            pltpu.sync_copy(buf_p.at[pl.ds(0, C)],
                            agg_sh.at[pl.ds(zbase + z * C, C)])
        if RPS % C:
            pltpu.sync_copy(buf_p.at[pl.ds(0, RPS % C)],
                            agg_sh.at[pl.ds(zbase + (RPS // C) * C, RPS % C)])

        @pl.when(sid == NS - 1)
        def _zero_tail():
            pltpu.sync_copy(buf_p.at[pl.ds(0, 16)],
                            agg_sh.at[pl.ds(NS * RPS, 16)])
        plsc.subcore_barrier()

        def chunk(k, carry):
            estart = base + k * C
            pltpu.sync_copy(src_hbm.at[pl.ds(estart, C)], idx_s)
            pltpu.sync_copy(dst_hbm.at[pl.ds(estart, C)], idx_d)
            pltpu.sync_copy(p_hbm.at[idx_s], buf_p)
            pltpu.sync_copy(q_hbm.at[idx_d], buf_q)
            pltpu.sync_copy(u_hbm.at[pl.ds(estart, C)], buf_u)

            def row(i, rcarry):
                for j in range(L // 16):
                    s = pl.ds(j * 16, 16)
                    v = buf_p[i, s] + buf_q[i, s] + buf_u[i, s]
                    buf_p[i, s] = jnp.maximum(v, 0.0)
                return rcarry
            lax.fori_loop(0, C, row, 0)

            if write_e:
                pltpu.sync_copy(buf_p, e_out.at[pl.ds(estart, C)])
            pltpu.sync_copy(buf_p, agg_sh.at[idx_d], add=True)
            return carry
        lax.fori_loop(0, NCHUNK, chunk, 0)

        plsc.subcore_barrier()
        pltpu.sync_copy(agg_sh.at[pl.ds(sid * RPS, RPS)],
                        agg_out.at[cid, pl.ds(sid * RPS, RPS)])

        @pl.when(sid == NS - 1)
        def _copy_tail():
            pltpu.sync_copy(agg_sh.at[pl.ds(NS * RPS, 16)],
                            agg_out.at[cid, pl.ds(NS * RPS, 16)])

    return sc_step


_sc_step_we = _make_sc_step(write_e=True)
_sc_step_ne = _make_sc_step(write_e=False)


def kernel(x, edge_index, edge_attr, W_ne, b_ne, W_ee, b_ee, W_e, b_e,
           W_n, b_n, W_d1, b_d1, W_d2, b_d2, W_r, b_r):
    src = edge_index[0].astype(jnp.int32)
    dst = edge_index[1].astype(jnp.int32)

    Wa0, Wb0, Wc0 = W_e[0, :L], W_e[0, L:2 * L], W_e[0, 2 * L:]
    Wa1, Wb1, Wc1 = W_e[1, :L], W_e[1, L:2 * L], W_e[1, 2 * L:]
    Wn0h, Wn0a = W_n[0, :L], W_n[0, L:]
    Wn1h, Wn1a = W_n[1, :L], W_n[1, L:]
    bne = b_ne.reshape(1, L)
    bee = b_ee.reshape(1, L)
    be0 = b_e[0].reshape(1, L)
    be1 = b_e[1].reshape(1, L)
    bn0 = b_n[0].reshape(1, L)
    bn1 = b_n[1].reshape(1, L)
    bd1 = b_d1.reshape(1, L)
    bd2 = b_d2.reshape(1, L)
    br = b_r.reshape(1, 1)

    h0, P0, Q0 = _node_encode(x, W_ne, bne, Wa0, Wb0)
    U0 = _edge_u0(edge_attr, W_ee, bee, Wc0, be0)
    e1, agg0 = _sc_step_we(P0, Q0, U0, src, dst)
    h1, P1, Q1 = _node_update(h0, agg0, Wn0h, Wn0a, bn0, Wa1, Wb1)
    U1 = _edge_u(e1, Wc1, be1)
    (agg1,) = _sc_step_ne(P1, Q1, U1, src, dst)
    out = _final(h1, agg1, Wn1h, Wn1a, bn1, W_d1, bd1, W_d2, bd2, W_r, br)
    return out
